# baseline (speedup 1.0000x reference)
import functools

import jax
import jax.numpy as jnp
from jax import lax
from jax.experimental import pallas as pl
from jax.experimental.pallas import tpu as pltpu
from jax.experimental.pallas import tpu_sc as plsc
from jax._src.pallas import mpmd as _mpmd

MOM = 0.99
EPS = 1e-12
RBR = 512
RBU = 256
DB = 4000


def _mega_body(lab_ref, feat_ref, g_ref, p_ref, dense_ref, nn_ref,
               d_s, fh_s, fl_s):
    i = pl.program_id(0)
    B = lab_ref.shape[1]
    nrk = B // RBR
    nup = B // RBU
    log_m = jnp.float32(jnp.log(MOM))

    x = p_ref[:, :]
    s = jnp.sum(x * x, axis=1, keepdims=True)
    dense_ref[:, :] = x * lax.rsqrt(jnp.maximum(s, EPS * EPS))

    @pl.when(i < nrk)
    def _rank():
        rows = lab_ref[0, pl.ds(i * RBR, RBR)].reshape(RBR, 1)
        eq = rows == lab_ref[:, :]
        j_idx = lax.broadcasted_iota(jnp.int32, (RBR, B), 1)
        i_idx = lax.broadcasted_iota(jnp.int32, (RBR, B), 0) + i * RBR
        contrib = jnp.where(eq,
                            jnp.where(j_idx > i_idx, 4097.0, 1.0), 0.0)
        S = jnp.sum(contrib, axis=1) - 1.0
        r = jnp.floor(S * (1.0 / 4096.0))
        cm1 = S - 4096.0 * r
        d_s[0, pl.ds(i * RBR, RBR)] = jnp.exp((cm1 + 1.0) * log_m)
        w = jnp.exp(r * log_m).reshape(RBR, 1)
        wf = w * feat_ref[:, :]
        fh = wf.astype(jnp.bfloat16)
        fh_s[pl.ds(i * RBR, RBR), :] = fh
        fl_s[pl.ds(i * RBR, RBR), :] = (wf - fh.astype(jnp.float32)
                                        ).astype(jnp.bfloat16)

    @pl.when((i >= nrk) & (i < nrk + nup))
    def _update():
        j = i - nrk
        rows = lab_ref[0, pl.ds(j * RBU, RBU)].reshape(RBU, 1)
        eq = jnp.where(rows == lab_ref[:, :], 1.0, 0.0
                       ).astype(jnp.bfloat16)
        dot = lambda a, b: lax.dot_general(
            a, b,
            dimension_numbers=(((1,), (0,)), ((), ())),
            preferred_element_type=jnp.float32,
        )
        sums = dot(eq, fh_s[:, :]) + dot(eq, fl_s[:, :])
        dec = d_s[0, pl.ds(j * RBU, RBU)].reshape(RBU, 1)
        new = dec * g_ref[:, :] + (1.0 - MOM) * sums
        s2 = jnp.sum(new * new, axis=1, keepdims=True)
        nn_ref[:, :] = new * lax.rsqrt(jnp.maximum(s2, EPS * EPS))


def _make_sc_gather(C, D, B):
    info = plsc.get_sparse_core_info()
    NC, NS = info.num_cores, info.num_subcores
    NW = NC * NS
    bpw = B // NW
    mesh = plsc.VectorSubcoreMesh(core_axis_name="c", subcore_axis_name="s")

    @functools.partial(
        pl.kernel,
        out_type=jax.ShapeDtypeStruct((B, D), jnp.float32),
        mesh=mesh,
        scratch_types=[
            pltpu.VMEM((bpw,), jnp.int32),
            pltpu.VMEM((bpw, D), jnp.float32),
            pltpu.SemaphoreType.DMA,
        ],
    )
    def gather(table_hbm, idx_hbm, out_hbm, idx_v, rows_v, sem):
        wid = lax.axis_index("s") * NC + lax.axis_index("c")
        base = wid * bpw
        pltpu.sync_copy(idx_hbm.at[pl.ds(base, bpw)], idx_v)
        pltpu.async_copy(table_hbm.at[idx_v], rows_v, sem).wait()
        pltpu.sync_copy(rows_v, out_hbm.at[pl.ds(base, bpw)])

    return gather


def _make_sc_scatter(C, D, B):
    info = plsc.get_sparse_core_info()
    NC, NS = info.num_cores, info.num_subcores
    NW = NC * NS
    bpw = B // NW
    mesh = plsc.VectorSubcoreMesh(core_axis_name="c", subcore_axis_name="s")

    def scatter_body(dense_hbm, idx_hbm, rows_hbm, out_hbm, idx_v, rows_v, sem):
        del dense_hbm
        wid = lax.axis_index("s") * NC + lax.axis_index("c")
        base = wid * bpw
        pltpu.sync_copy(idx_hbm.at[pl.ds(base, bpw)], idx_v)
        pltpu.sync_copy(rows_hbm.at[pl.ds(base, bpw)], rows_v)
        pltpu.async_copy(rows_v, out_hbm.at[idx_v], sem).wait()

    return _mpmd._mpmd_map(
        [(mesh, scatter_body)],
        jax.ShapeDtypeStruct((C, D), jnp.float32),
        input_output_aliases={0: 0},
        scratch_types=[
            pltpu.VMEM((bpw,), jnp.int32),
            pltpu.VMEM((bpw, D), jnp.float32),
            pltpu.SemaphoreType.DMA,
        ],
    )


def kernel(features, labels, prototypes):
    B, D = features.shape
    C = prototypes.shape[0]
    nrk = B // RBR
    nup = B // RBU
    lab2d = labels.reshape(1, B)

    gathered = _make_sc_gather(C, D, B)(prototypes, labels)

    dense, nn = pl.pallas_call(
        _mega_body,
        grid=(C // DB,),
        in_specs=[
            pl.BlockSpec((1, B), lambda i: (0, 0)),
            pl.BlockSpec((RBR, D), lambda i: (jnp.clip(i, 0, nrk - 1), 0)),
            pl.BlockSpec((RBU, D),
                         lambda i: (jnp.clip(i - nrk, 0, nup - 1), 0)),
            pl.BlockSpec((DB, D), lambda i: (i, 0)),
        ],
        out_specs=[
            pl.BlockSpec((DB, D), lambda i: (i, 0)),
            pl.BlockSpec((RBU, D),
                         lambda i: (jnp.clip(i - nrk, 0, nup - 1), 0)),
        ],
        out_shape=[
            jax.ShapeDtypeStruct((C, D), jnp.float32),
            jax.ShapeDtypeStruct((B, D), jnp.float32),
        ],
        scratch_shapes=[
            pltpu.VMEM((1, B), jnp.float32),
            pltpu.VMEM((B, D), jnp.bfloat16),
            pltpu.VMEM((B, D), jnp.bfloat16),
        ],
    )(lab2d, features, gathered, prototypes)

    return _make_sc_scatter(C, D, B)(dense, labels, nn)


# speedup vs baseline: 1.0278x; 1.0278x over previous
import functools

import jax
import jax.numpy as jnp
from jax import lax
from jax.experimental import pallas as pl
from jax.experimental.pallas import tpu as pltpu
from jax.experimental.pallas import tpu_sc as plsc
from jax._src.pallas import mpmd as _mpmd

MOM = 0.99
EPS = 1e-12
RBR = 512
RBU = 256
DB = 4000


def _mega_body(lab_ref, feat_ref, g_ref, p_ref, dense_ref, nn_ref,
               d_s, fh_s):
    i = pl.program_id(0)
    B = lab_ref.shape[1]
    nrk = B // RBR
    nup = B // RBU
    log_m = jnp.float32(jnp.log(MOM))

    x = p_ref[:, :]
    s = jnp.sum(x * x, axis=1, keepdims=True)
    dense_ref[:, :] = x * lax.rsqrt(jnp.maximum(s, EPS * EPS))

    @pl.when(i < nrk)
    def _rank():
        rows = lab_ref[0, pl.ds(i * RBR, RBR)].reshape(RBR, 1)
        eq = rows == lab_ref[:, :]
        j_idx = lax.broadcasted_iota(jnp.int32, (RBR, B), 1)
        i_idx = lax.broadcasted_iota(jnp.int32, (RBR, B), 0) + i * RBR
        contrib = jnp.where(eq,
                            jnp.where(j_idx > i_idx, 4097.0, 1.0), 0.0)
        S = jnp.sum(contrib, axis=1) - 1.0
        r = jnp.floor(S * (1.0 / 4096.0))
        cm1 = S - 4096.0 * r
        d_s[0, pl.ds(i * RBR, RBR)] = jnp.exp((cm1 + 1.0) * log_m)
        w = jnp.exp(r * log_m).reshape(RBR, 1)
        wf = w * feat_ref[:, :]
        fh_s[pl.ds(i * RBR, RBR), :] = wf.astype(jnp.bfloat16)

    @pl.when((i >= nrk) & (i < nrk + nup))
    def _update():
        j = i - nrk
        rows = lab_ref[0, pl.ds(j * RBU, RBU)].reshape(RBU, 1)
        eq = jnp.where(rows == lab_ref[:, :], 1.0, 0.0
                       ).astype(jnp.bfloat16)
        dot = lambda a, b: lax.dot_general(
            a, b,
            dimension_numbers=(((1,), (0,)), ((), ())),
            preferred_element_type=jnp.float32,
        )
        sums = dot(eq, fh_s[:, :])
        dec = d_s[0, pl.ds(j * RBU, RBU)].reshape(RBU, 1)
        new = dec * g_ref[:, :] + (1.0 - MOM) * sums
        s2 = jnp.sum(new * new, axis=1, keepdims=True)
        nn_ref[:, :] = new * lax.rsqrt(jnp.maximum(s2, EPS * EPS))


def _make_sc_gather(C, D, B):
    info = plsc.get_sparse_core_info()
    NC, NS = info.num_cores, info.num_subcores
    NW = NC * NS
    bpw = B // NW
    mesh = plsc.VectorSubcoreMesh(core_axis_name="c", subcore_axis_name="s")

    @functools.partial(
        pl.kernel,
        out_type=jax.ShapeDtypeStruct((B, D), jnp.float32),
        mesh=mesh,
        scratch_types=[
            pltpu.VMEM((bpw,), jnp.int32),
            pltpu.VMEM((bpw, D), jnp.float32),
            pltpu.SemaphoreType.DMA,
        ],
    )
    def gather(table_hbm, idx_hbm, out_hbm, idx_v, rows_v, sem):
        wid = lax.axis_index("s") * NC + lax.axis_index("c")
        base = wid * bpw
        pltpu.sync_copy(idx_hbm.at[pl.ds(base, bpw)], idx_v)
        pltpu.async_copy(table_hbm.at[idx_v], rows_v, sem).wait()
        pltpu.sync_copy(rows_v, out_hbm.at[pl.ds(base, bpw)])

    return gather


def _make_sc_scatter(C, D, B):
    info = plsc.get_sparse_core_info()
    NC, NS = info.num_cores, info.num_subcores
    NW = NC * NS
    bpw = B // NW
    mesh = plsc.VectorSubcoreMesh(core_axis_name="c", subcore_axis_name="s")

    def scatter_body(dense_hbm, idx_hbm, rows_hbm, out_hbm, idx_v, rows_v, sem):
        del dense_hbm
        wid = lax.axis_index("s") * NC + lax.axis_index("c")
        base = wid * bpw
        pltpu.sync_copy(idx_hbm.at[pl.ds(base, bpw)], idx_v)
        pltpu.sync_copy(rows_hbm.at[pl.ds(base, bpw)], rows_v)
        pltpu.async_copy(rows_v, out_hbm.at[idx_v], sem).wait()

    return _mpmd._mpmd_map(
        [(mesh, scatter_body)],
        jax.ShapeDtypeStruct((C, D), jnp.float32),
        input_output_aliases={0: 0},
        scratch_types=[
            pltpu.VMEM((bpw,), jnp.int32),
            pltpu.VMEM((bpw, D), jnp.float32),
            pltpu.SemaphoreType.DMA,
        ],
    )


def kernel(features, labels, prototypes):
    B, D = features.shape
    C = prototypes.shape[0]
    nrk = B // RBR
    nup = B // RBU
    lab2d = labels.reshape(1, B)

    gathered = _make_sc_gather(C, D, B)(prototypes, labels)

    dense, nn = pl.pallas_call(
        _mega_body,
        grid=(C // DB,),
        in_specs=[
            pl.BlockSpec((1, B), lambda i: (0, 0)),
            pl.BlockSpec((RBR, D), lambda i: (jnp.clip(i, 0, nrk - 1), 0)),
            pl.BlockSpec((RBU, D),
                         lambda i: (jnp.clip(i - nrk, 0, nup - 1), 0)),
            pl.BlockSpec((DB, D), lambda i: (i, 0)),
        ],
        out_specs=[
            pl.BlockSpec((DB, D), lambda i: (i, 0)),
            pl.BlockSpec((RBU, D),
                         lambda i: (jnp.clip(i - nrk, 0, nup - 1), 0)),
        ],
        out_shape=[
            jax.ShapeDtypeStruct((C, D), jnp.float32),
            jax.ShapeDtypeStruct((B, D), jnp.float32),
        ],
        scratch_shapes=[
            pltpu.VMEM((1, B), jnp.float32),
            pltpu.VMEM((B, D), jnp.bfloat16),
        ],
    )(lab2d, features, gathered, prototypes)

    return _make_sc_scatter(C, D, B)(dense, labels, nn)
